# Initial kernel scaffold; baseline (speedup 1.0000x reference)
import jax
import jax.numpy as jnp
from jax.experimental import pallas as pl


def kernel(x, block1_edges, block2_edges, pos_edges, neg_edges, W1, b1, W2, b2):
    raise NotImplementedError("write your pallas kernel here")


# speedup vs baseline: 3.3898x; 3.3898x over previous
import functools

import jax
import jax.numpy as jnp
from jax import lax
from jax.experimental import pallas as pl
from jax.experimental.pallas import tpu as pltpu
from jax.experimental.pallas import tpu_sc as plsc

N = 10000
NP = 10240
R = 3
E = 160000
D = 256
H = 128
NT = 16
NC = 2
IW = 100
ROWS = E // IW
TROWS = ROWS // NT
CR = 2
NCHUNK = TROWS // CR
RPT = NP // NT

_mesh = plsc.VectorSubcoreMesh(core_axis_name="c", subcore_axis_name="s")


DROWS = 125
DCHUNK = 5


@functools.partial(
    pl.kernel,
    out_type=jax.ShapeDtypeStruct((NC, NT, 6 * NP), jnp.float32),
    mesh=_mesh,
    compiler_params=pltpu.CompilerParams(needs_layout_passes=False),
    scratch_types=[
        pltpu.VMEM((6 * NP,), jnp.float32),
        pltpu.VMEM((DROWS, 16), jnp.int32),
        pltpu.SemaphoreType.DMA,
    ],
)
def _deg_kernel(b1e, b2e, deg_out, hist, idx_v, sem):
    c = lax.axis_index("c")
    s = lax.axis_index("s")

    def zbody(i, _):
        hist[pl.ds(i * 16, 16)] = jnp.zeros((16,), jnp.float32)
        return _
    lax.fori_loop(0, (6 * NP) // 16, zbody, None)

    ones16 = jnp.ones((16,), jnp.float32)

    def histogram(edges):
        for a in range(6):
            aoff = jnp.full((16,), a * NP, jnp.int32)

            def chunk_body(k, _, a=a, aoff=aoff):
                pltpu.sync_copy(edges.at[a // 2, a % 2, s, k], idx_v)
                for i in range(DROWS):
                    v = idx_v[i, pl.ds(0, 16)]
                    plsc.addupdate_scatter(hist, [v + aoff], ones16)
                return _
            lax.fori_loop(0, DCHUNK, chunk_body, None)

    @pl.when(c == 0)
    def _():
        histogram(b1e)

    @pl.when(c == 1)
    def _():
        histogram(b2e)

    pltpu.sync_copy(hist, deg_out.at[c, s])


@functools.partial(
    pl.kernel,
    out_type=jax.ShapeDtypeStruct((NC, R, NP, H), jnp.float32),
    mesh=_mesh,
    scratch_types=[
        pltpu.VMEM_SHARED((NP, H), jnp.float32),
        pltpu.VMEM((CR, IW), jnp.int32),
        pltpu.VMEM((CR, IW), jnp.int32),
        pltpu.VMEM((CR * IW, H), jnp.float32),
        pltpu.SemaphoreType.DMA,
    ],
)
def _agg_kernel(hs, edges, zeros_hbm, agg_out, acc, src_i, dst_i, rows_v, sem):
    c = lax.axis_index("c")
    s = lax.axis_index("s")

    for r in range(R):
        pltpu.sync_copy(zeros_hbm, acc.at[pl.ds(s * RPT, RPT)])
        plsc.subcore_barrier()

        def chunk_body(k, _):
            pltpu.sync_copy(edges.at[r, 0, s, k], src_i)
            pltpu.sync_copy(edges.at[r, 1, s, k], dst_i)
            descs = [
                pltpu.async_copy(
                    hs.at[c, r].at[src_i.at[j]],
                    rows_v.at[pl.ds(j * IW, IW)], sem)
                for j in range(CR)
            ]
            for dsc in descs:
                dsc.wait()
            for j in range(CR):
                pltpu.sync_copy(rows_v.at[pl.ds(j * IW, IW)],
                                acc.at[dst_i.at[j]], add=True)
            return _
        lax.fori_loop(0, NCHUNK, chunk_body, None)

        plsc.subcore_barrier()
        pltpu.sync_copy(acc.at[pl.ds(s * RPT, RPT)],
                        agg_out.at[c, r, pl.ds(s * RPT, RPT)])


@functools.partial(
    pl.kernel,
    out_type=[jax.ShapeDtypeStruct((NC, E, H), jnp.float32)
              for _ in range(4)],
    mesh=_mesh,
    scratch_types=[
        pltpu.VMEM((CR, IW), jnp.int32),
        pltpu.VMEM((CR * IW, H), jnp.float32),
        pltpu.SemaphoreType.DMA,
    ],
)
def _score_gather_kernel(h2s, pe, ne, pu, pv, nu, nv, idx_v, rows_v, sem):
    c = lax.axis_index("c")
    s = lax.axis_index("s")

    for eref, io, outref in ((pe, 0, pu), (pe, 1, pv),
                             (ne, 0, nu), (ne, 1, nv)):
        def chunk_body(k, _, eref=eref, io=io, outref=outref):
            pltpu.sync_copy(eref.at[io, s, k], idx_v)
            descs = [
                pltpu.async_copy(
                    h2s.at[c].at[idx_v.at[j]],
                    rows_v.at[pl.ds(j * IW, IW)], sem)
                for j in range(CR)
            ]
            for dsc in descs:
                dsc.wait()
            base = (s * NCHUNK + k) * (CR * IW)
            pltpu.sync_copy(rows_v, outref.at[c, pl.ds(base, CR * IW)])
            return _
        lax.fori_loop(0, NCHUNK, chunk_body, None)


NB = 256
GRID = NP // NB


def _tc_a_body(x_ref, deg_ref, w1_ref, hs_ref, rs_ref):
    dsum = jnp.sum(deg_ref[...], axis=1)
    deg = jnp.concatenate([dsum[0], dsum[1]], axis=0)
    rs = lax.rsqrt(jnp.maximum(deg, 1.0))
    rs_ref[...] = rs
    x = x_ref[...]
    for r in range(R):
        xs = x * rs[2 * r][:, None]
        t = jnp.dot(xs, w1_ref[r], preferred_element_type=jnp.float32)
        hs_ref[0, r] = t[:, :H]
        hs_ref[1, r] = t[:, H:]


def _tc_b_body(agg_ref, rs_ref, b1_ref, w2_ref, hs2_ref):
    rs = rs_ref[...]
    bsum = b1_ref[0] + b1_ref[1] + b1_ref[2]
    h = jnp.broadcast_to(bsum[None, :], (NB, D))
    for r in range(R):
        ar = jnp.concatenate([agg_ref[0, r], agg_ref[1, r]], axis=1)
        h = h + ar * rs[2 * r + 1][:, None]
    h = jnp.maximum(h, 0.0)
    for r in range(R):
        hsr = h * rs[6 + 2 * r][:, None]
        t = jnp.dot(hsr, w2_ref[r], preferred_element_type=jnp.float32)
        hs2_ref[0, r] = t[:, :H]
        hs2_ref[1, r] = t[:, H:]


def _tc_c_body(agg_ref, rs_ref, b2_ref, h2s_ref):
    rs = rs_ref[...]
    bsum = b2_ref[0] + b2_ref[1] + b2_ref[2]
    h = jnp.broadcast_to(bsum[None, :], (NB, D))
    for r in range(R):
        ar = jnp.concatenate([agg_ref[0, r], agg_ref[1, r]], axis=1)
        h = h + ar * rs[7 + 2 * r][:, None]
    h2s_ref[0] = h[:, :H]
    h2s_ref[1] = h[:, H:]


BE = 4000
DGRID = E // BE
DCOL = 500


def _tc_d_body(pu_ref, pv_ref, nu_ref, nv_ref, pos_ref, neg_ref):
    pos_ref[...] = jnp.sum(pu_ref[0] * pv_ref[0] + pu_ref[1] * pv_ref[1],
                           axis=1).reshape(BE // DCOL, DCOL)
    neg_ref[...] = jnp.sum(nu_ref[0] * nv_ref[0] + nu_ref[1] * nv_ref[1],
                           axis=1).reshape(BE // DCOL, DCOL)


def kernel(x, block1_edges, block2_edges, pos_edges, neg_edges, W1, b1, W2, b2):
    f32 = jnp.float32
    xp = jnp.pad(x, ((0, NP - N), (0, 0)))
    b1d = block1_edges.reshape(R, 2, NT, DCHUNK, DROWS, 16)
    b2d = block2_edges.reshape(R, 2, NT, DCHUNK, DROWS, 16)
    b1r = block1_edges.reshape(R, 2, NT, NCHUNK, CR, IW)
    b2r = block2_edges.reshape(R, 2, NT, NCHUNK, CR, IW)
    per = pos_edges.reshape(2, NT, NCHUNK, CR, IW)
    ner = neg_edges.reshape(2, NT, NCHUNK, CR, IW)
    zeros_hbm = jnp.zeros((RPT, H), f32)

    deg = _deg_kernel(b1d, b2d).reshape(NC, NT, 6, NP)

    hs1, rs = pl.pallas_call(
        _tc_a_body,
        grid=(GRID,),
        in_specs=[
            pl.BlockSpec((NB, D), lambda i: (i, 0)),
            pl.BlockSpec((NC, NT, 6, NB), lambda i: (0, 0, 0, i)),
            pl.BlockSpec((R, D, D), lambda i: (0, 0, 0)),
        ],
        out_specs=[
            pl.BlockSpec((NC, R, NB, H), lambda i: (0, 0, i, 0)),
            pl.BlockSpec((12, NB), lambda i: (0, i)),
        ],
        out_shape=[
            jax.ShapeDtypeStruct((NC, R, NP, H), f32),
            jax.ShapeDtypeStruct((12, NP), f32),
        ],
    )(xp, deg, W1)

    agg1 = _agg_kernel(hs1, b1r, zeros_hbm)

    hs2 = pl.pallas_call(
        _tc_b_body,
        grid=(GRID,),
        in_specs=[
            pl.BlockSpec((NC, R, NB, H), lambda i: (0, 0, i, 0)),
            pl.BlockSpec((12, NB), lambda i: (0, i)),
            pl.BlockSpec((R, D), lambda i: (0, 0)),
            pl.BlockSpec((R, D, D), lambda i: (0, 0, 0)),
        ],
        out_specs=pl.BlockSpec((NC, R, NB, H), lambda i: (0, 0, i, 0)),
        out_shape=jax.ShapeDtypeStruct((NC, R, NP, H), f32),
    )(agg1, rs, b1, W2)

    agg2 = _agg_kernel(hs2, b2r, zeros_hbm)

    h2s = pl.pallas_call(
        _tc_c_body,
        grid=(GRID,),
        in_specs=[
            pl.BlockSpec((NC, R, NB, H), lambda i: (0, 0, i, 0)),
            pl.BlockSpec((12, NB), lambda i: (0, i)),
            pl.BlockSpec((R, D), lambda i: (0, 0)),
        ],
        out_specs=pl.BlockSpec((NC, NB, H), lambda i: (0, i, 0)),
        out_shape=jax.ShapeDtypeStruct((NC, NP, H), f32),
    )(agg2, rs, b2)

    pu, pv, nu, nv = _score_gather_kernel(h2s, per, ner)

    pos, neg = pl.pallas_call(
        _tc_d_body,
        grid=(DGRID,),
        in_specs=[pl.BlockSpec((NC, BE, H), lambda i: (0, i, 0))
                  for _ in range(4)],
        out_specs=[pl.BlockSpec((BE // DCOL, DCOL), lambda i: (i, 0))
                   for _ in range(2)],
        out_shape=[jax.ShapeDtypeStruct((E // DCOL, DCOL), f32)
                   for _ in range(2)],
    )(pu, pv, nu, nv)

    return (pos.reshape(E, 1), neg.reshape(E, 1))


# speedup vs baseline: 3.7727x; 1.1129x over previous
import functools

import jax
import jax.numpy as jnp
from jax import lax
from jax.experimental import pallas as pl
from jax.experimental.pallas import tpu as pltpu
from jax.experimental.pallas import tpu_sc as plsc

N = 10000
NP = 10240
R = 3
E = 160000
D = 256
H = 128
NT = 16
NC = 2
IW = 100
ROWS = E // IW
TROWS = ROWS // NT
CR = 2
NCHUNK = TROWS // CR
RPT = NP // NT

_mesh = plsc.VectorSubcoreMesh(core_axis_name="c", subcore_axis_name="s")


DROWS = 125
DCHUNK = 5


@functools.partial(
    pl.kernel,
    out_type=jax.ShapeDtypeStruct((NC, NT, 6 * NP), jnp.float32),
    mesh=_mesh,
    compiler_params=pltpu.CompilerParams(needs_layout_passes=False),
    scratch_types=[
        pltpu.VMEM((6 * NP,), jnp.float32),
        pltpu.VMEM((DROWS, 16), jnp.int32),
        pltpu.SemaphoreType.DMA,
    ],
)
def _deg_kernel(b1e, b2e, deg_out, hist, idx_v, sem):
    c = lax.axis_index("c")
    s = lax.axis_index("s")

    def zbody(i, _):
        hist[pl.ds(i * 16, 16)] = jnp.zeros((16,), jnp.float32)
        return _
    lax.fori_loop(0, (6 * NP) // 16, zbody, None)

    ones16 = jnp.ones((16,), jnp.float32)

    def histogram(edges):
        for a in range(6):
            aoff = jnp.full((16,), a * NP, jnp.int32)

            def chunk_body(k, _, a=a, aoff=aoff):
                pltpu.sync_copy(edges.at[a // 2, a % 2, s, k], idx_v)
                for i in range(DROWS):
                    v = idx_v[i, pl.ds(0, 16)]
                    plsc.addupdate_scatter(hist, [v + aoff], ones16)
                return _
            lax.fori_loop(0, DCHUNK, chunk_body, None)

    @pl.when(c == 0)
    def _():
        histogram(b1e)

    @pl.when(c == 1)
    def _():
        histogram(b2e)

    pltpu.sync_copy(hist, deg_out.at[c, s])


IW2 = 80
NCH2 = (E // NT) // IW2


@functools.partial(
    pl.kernel,
    out_type=jax.ShapeDtypeStruct((NC, R, NP, H), jnp.float32),
    mesh=_mesh,
    scratch_types=[
        pltpu.VMEM_SHARED((NP, H), jnp.float32),
        pltpu.VMEM((1, IW2), jnp.int32),
        pltpu.VMEM((1, IW2), jnp.int32),
        pltpu.VMEM((1, IW2), jnp.int32),
        pltpu.VMEM((1, IW2), jnp.int32),
        pltpu.VMEM((IW2, H), jnp.float32),
        pltpu.VMEM((IW2, H), jnp.float32),
        pltpu.SemaphoreType.DMA,
        pltpu.SemaphoreType.DMA,
        pltpu.SemaphoreType.DMA,
        pltpu.SemaphoreType.DMA,
    ],
)
def _agg_kernel(hs, edges, zeros_hbm, agg_out, acc,
                src0, dst0, src1, dst1, rows0, rows1,
                gsem0, gsem1, ssem0, ssem1):
    c = lax.axis_index("c")
    s = lax.axis_index("s")

    for r in range(R):
        pltpu.sync_copy(zeros_hbm, acc.at[pl.ds(s * RPT, RPT)])
        plsc.subcore_barrier()

        def pair_body(i, _):
            k0 = 2 * i
            k1 = 2 * i + 1

            @pl.when(i > 0)
            def _():
                pltpu.make_async_copy(
                    zeros_hbm.at[pl.ds(0, IW2)], rows0, ssem0).wait()
            pltpu.sync_copy(edges.at[r, 0, s, k0], src0)
            pltpu.sync_copy(edges.at[r, 1, s, k0], dst0)
            g0 = pltpu.async_copy(hs.at[c, r].at[src0.at[0]], rows0, gsem0)

            @pl.when(i > 0)
            def _():
                pltpu.make_async_copy(
                    zeros_hbm.at[pl.ds(0, IW2)], rows1, ssem1).wait()
            pltpu.sync_copy(edges.at[r, 0, s, k1], src1)
            pltpu.sync_copy(edges.at[r, 1, s, k1], dst1)
            g1 = pltpu.async_copy(hs.at[c, r].at[src1.at[0]], rows1, gsem1)

            g0.wait()
            pltpu.async_copy(rows0, acc.at[dst0.at[0]], ssem0, add=True)
            g1.wait()
            pltpu.async_copy(rows1, acc.at[dst1.at[0]], ssem1, add=True)
            return _
        lax.fori_loop(0, NCH2 // 2, pair_body, None)

        pltpu.make_async_copy(zeros_hbm.at[pl.ds(0, IW2)], rows0, ssem0).wait()
        pltpu.sync_copy(edges.at[r, 0, s, NCH2 - 1], src0)
        pltpu.sync_copy(edges.at[r, 1, s, NCH2 - 1], dst0)
        pltpu.async_copy(hs.at[c, r].at[src0.at[0]], rows0, gsem0).wait()
        pltpu.async_copy(rows0, acc.at[dst0.at[0]], ssem0, add=True)

        pltpu.make_async_copy(zeros_hbm.at[pl.ds(0, IW2)], rows0, ssem0).wait()
        pltpu.make_async_copy(zeros_hbm.at[pl.ds(0, IW2)], rows1, ssem1).wait()

        plsc.subcore_barrier()
        pltpu.sync_copy(acc.at[pl.ds(s * RPT, RPT)],
                        agg_out.at[c, r, pl.ds(s * RPT, RPT)])


@functools.partial(
    pl.kernel,
    out_type=[jax.ShapeDtypeStruct((NC, E, H), jnp.float32)
              for _ in range(4)],
    mesh=_mesh,
    scratch_types=[
        pltpu.VMEM((CR, IW), jnp.int32),
        pltpu.VMEM((CR, IW), jnp.int32),
        pltpu.VMEM((CR * IW, H), jnp.float32),
        pltpu.VMEM((CR * IW, H), jnp.float32),
        pltpu.SemaphoreType.DMA,
        pltpu.SemaphoreType.DMA,
        pltpu.SemaphoreType.DMA,
        pltpu.SemaphoreType.DMA,
    ],
)
def _score_gather_kernel(h2s, pe, ne, zeros_hbm, pu, pv, nu, nv,
                         idx0, idx1, rows0, rows1,
                         gsem0, gsem1, wsem0, wsem1):
    c = lax.axis_index("c")
    s = lax.axis_index("s")
    CB = CR * IW

    for eref, io, outref in ((pe, 0, pu), (pe, 1, pv),
                             (ne, 0, nu), (ne, 1, nv)):
        def pair_body(i, _, eref=eref, io=io, outref=outref):
            k0 = 2 * i
            k1 = 2 * i + 1

            @pl.when(i > 0)
            def _():
                pltpu.make_async_copy(
                    zeros_hbm.at[pl.ds(0, CB)], rows0, wsem0).wait()
            pltpu.sync_copy(eref.at[io, s, k0], idx0)
            d0 = [pltpu.async_copy(h2s.at[c].at[idx0.at[j]],
                                   rows0.at[pl.ds(j * IW, IW)], gsem0)
                  for j in range(CR)]

            @pl.when(i > 0)
            def _():
                pltpu.make_async_copy(
                    zeros_hbm.at[pl.ds(0, CB)], rows1, wsem1).wait()
            pltpu.sync_copy(eref.at[io, s, k1], idx1)
            d1 = [pltpu.async_copy(h2s.at[c].at[idx1.at[j]],
                                   rows1.at[pl.ds(j * IW, IW)], gsem1)
                  for j in range(CR)]

            for d in d0:
                d.wait()
            pltpu.async_copy(
                rows0, outref.at[c, pl.ds((s * NCHUNK + k0) * CB, CB)], wsem0)
            for d in d1:
                d.wait()
            pltpu.async_copy(
                rows1, outref.at[c, pl.ds((s * NCHUNK + k1) * CB, CB)], wsem1)
            return _
        lax.fori_loop(0, NCHUNK // 2, pair_body, None)

        pltpu.make_async_copy(zeros_hbm.at[pl.ds(0, CB)], rows0, wsem0).wait()
        pltpu.make_async_copy(zeros_hbm.at[pl.ds(0, CB)], rows1, wsem1).wait()


NB = 256
GRID = NP // NB


def _tc_a_body(x_ref, deg_ref, w1_ref, hs_ref, rs_ref):
    dsum = jnp.sum(deg_ref[...], axis=1)
    deg = jnp.concatenate([dsum[0], dsum[1]], axis=0)
    rs = lax.rsqrt(jnp.maximum(deg, 1.0))
    rs_ref[...] = rs
    x = x_ref[...]
    for r in range(R):
        xs = x * rs[2 * r][:, None]
        t = jnp.dot(xs, w1_ref[r], preferred_element_type=jnp.float32)
        hs_ref[0, r] = t[:, :H]
        hs_ref[1, r] = t[:, H:]


def _tc_b_body(agg_ref, rs_ref, b1_ref, w2_ref, hs2_ref):
    rs = rs_ref[...]
    bsum = b1_ref[0] + b1_ref[1] + b1_ref[2]
    h = jnp.broadcast_to(bsum[None, :], (NB, D))
    for r in range(R):
        ar = jnp.concatenate([agg_ref[0, r], agg_ref[1, r]], axis=1)
        h = h + ar * rs[2 * r + 1][:, None]
    h = jnp.maximum(h, 0.0)
    for r in range(R):
        hsr = h * rs[6 + 2 * r][:, None]
        t = jnp.dot(hsr, w2_ref[r], preferred_element_type=jnp.float32)
        hs2_ref[0, r] = t[:, :H]
        hs2_ref[1, r] = t[:, H:]


def _tc_c_body(agg_ref, rs_ref, b2_ref, h2s_ref):
    rs = rs_ref[...]
    bsum = b2_ref[0] + b2_ref[1] + b2_ref[2]
    h = jnp.broadcast_to(bsum[None, :], (NB, D))
    for r in range(R):
        ar = jnp.concatenate([agg_ref[0, r], agg_ref[1, r]], axis=1)
        h = h + ar * rs[7 + 2 * r][:, None]
    h2s_ref[0] = h[:, :H]
    h2s_ref[1] = h[:, H:]


BE = 4000
DGRID = E // BE
DCOL = 500


def _tc_d_body(pu_ref, pv_ref, nu_ref, nv_ref, pos_ref, neg_ref):
    pos_ref[...] = jnp.sum(pu_ref[0] * pv_ref[0] + pu_ref[1] * pv_ref[1],
                           axis=1).reshape(BE // DCOL, DCOL)
    neg_ref[...] = jnp.sum(nu_ref[0] * nv_ref[0] + nu_ref[1] * nv_ref[1],
                           axis=1).reshape(BE // DCOL, DCOL)


def kernel(x, block1_edges, block2_edges, pos_edges, neg_edges, W1, b1, W2, b2):
    f32 = jnp.float32
    xp = jnp.pad(x, ((0, NP - N), (0, 0)))
    b1d = block1_edges.reshape(R, 2, NT, DCHUNK, DROWS, 16)
    b2d = block2_edges.reshape(R, 2, NT, DCHUNK, DROWS, 16)
    b1r = block1_edges.reshape(R, 2, NT, NCH2, 1, IW2)
    b2r = block2_edges.reshape(R, 2, NT, NCH2, 1, IW2)
    assert NT * NCH2 * IW2 == E
    per = pos_edges.reshape(2, NT, NCHUNK, CR, IW)
    ner = neg_edges.reshape(2, NT, NCHUNK, CR, IW)
    zeros_hbm = jnp.zeros((RPT, H), f32)

    deg = _deg_kernel(b1d, b2d).reshape(NC, NT, 6, NP)

    hs1, rs = pl.pallas_call(
        _tc_a_body,
        grid=(GRID,),
        in_specs=[
            pl.BlockSpec((NB, D), lambda i: (i, 0)),
            pl.BlockSpec((NC, NT, 6, NB), lambda i: (0, 0, 0, i)),
            pl.BlockSpec((R, D, D), lambda i: (0, 0, 0)),
        ],
        out_specs=[
            pl.BlockSpec((NC, R, NB, H), lambda i: (0, 0, i, 0)),
            pl.BlockSpec((12, NB), lambda i: (0, i)),
        ],
        out_shape=[
            jax.ShapeDtypeStruct((NC, R, NP, H), f32),
            jax.ShapeDtypeStruct((12, NP), f32),
        ],
    )(xp, deg, W1)

    agg1 = _agg_kernel(hs1, b1r, zeros_hbm)

    hs2 = pl.pallas_call(
        _tc_b_body,
        grid=(GRID,),
        in_specs=[
            pl.BlockSpec((NC, R, NB, H), lambda i: (0, 0, i, 0)),
            pl.BlockSpec((12, NB), lambda i: (0, i)),
            pl.BlockSpec((R, D), lambda i: (0, 0)),
            pl.BlockSpec((R, D, D), lambda i: (0, 0, 0)),
        ],
        out_specs=pl.BlockSpec((NC, R, NB, H), lambda i: (0, 0, i, 0)),
        out_shape=jax.ShapeDtypeStruct((NC, R, NP, H), f32),
    )(agg1, rs, b1, W2)

    agg2 = _agg_kernel(hs2, b2r, zeros_hbm)

    h2s = pl.pallas_call(
        _tc_c_body,
        grid=(GRID,),
        in_specs=[
            pl.BlockSpec((NC, R, NB, H), lambda i: (0, 0, i, 0)),
            pl.BlockSpec((12, NB), lambda i: (0, i)),
            pl.BlockSpec((R, D), lambda i: (0, 0)),
        ],
        out_specs=pl.BlockSpec((NC, NB, H), lambda i: (0, i, 0)),
        out_shape=jax.ShapeDtypeStruct((NC, NP, H), f32),
    )(agg2, rs, b2)

    pu, pv, nu, nv = _score_gather_kernel(h2s, per, ner, zeros_hbm)

    pos, neg = pl.pallas_call(
        _tc_d_body,
        grid=(DGRID,),
        in_specs=[pl.BlockSpec((NC, BE, H), lambda i: (0, i, 0))
                  for _ in range(4)],
        out_specs=[pl.BlockSpec((BE // DCOL, DCOL), lambda i: (i, 0))
                   for _ in range(2)],
        out_shape=[jax.ShapeDtypeStruct((E // DCOL, DCOL), f32)
                   for _ in range(2)],
    )(pu, pv, nu, nv)

    return (pos.reshape(E, 1), neg.reshape(E, 1))


# speedup vs baseline: 3.7740x; 1.0004x over previous
import functools

import jax
import jax.numpy as jnp
from jax import lax
from jax.experimental import pallas as pl
from jax.experimental.pallas import tpu as pltpu
from jax.experimental.pallas import tpu_sc as plsc

N = 10000
NP = 10240
R = 3
E = 160000
D = 256
H = 128
NT = 16
NC = 2
IW = 100
CR = 2
NCHUNK = (E // NT) // (CR * IW)
RPT = NP // NT

_mesh = plsc.VectorSubcoreMesh(core_axis_name="c", subcore_axis_name="s")


DROWS = 125
DCHUNK = 5


@functools.partial(
    pl.kernel,
    out_type=jax.ShapeDtypeStruct((NC, NT, 6 * NP), jnp.float32),
    mesh=_mesh,
    compiler_params=pltpu.CompilerParams(needs_layout_passes=False),
    scratch_types=[
        pltpu.VMEM((6 * NP,), jnp.float32),
        pltpu.VMEM((DROWS, 16), jnp.int32),
        pltpu.SemaphoreType.DMA,
    ],
)
def _deg_kernel(b1e, b2e, deg_out, hist, idx_v, sem):
    c = lax.axis_index("c")
    s = lax.axis_index("s")

    def zbody(i, _):
        hist[pl.ds(i * 16, 16)] = jnp.zeros((16,), jnp.float32)
        return _
    lax.fori_loop(0, (6 * NP) // 16, zbody, None)

    ones16 = jnp.ones((16,), jnp.float32)

    def histogram(edges):
        for a in range(6):
            aoff = jnp.full((16,), a * NP, jnp.int32)

            def chunk_body(k, _, a=a, aoff=aoff):
                pltpu.sync_copy(edges.at[a // 2, a % 2, s, k], idx_v)
                for i in range(DROWS):
                    v = idx_v[i, pl.ds(0, 16)]
                    plsc.addupdate_scatter(hist, [v + aoff], ones16)
                return _
            lax.fori_loop(0, DCHUNK, chunk_body, None)

    @pl.when(c == 0)
    def _():
        histogram(b1e)

    @pl.when(c == 1)
    def _():
        histogram(b2e)

    pltpu.sync_copy(hist, deg_out.at[c, s])


IW2 = 80
NCH2 = (E // NT) // IW2


@functools.partial(
    pl.kernel,
    out_type=jax.ShapeDtypeStruct((NC, R, NP, H), jnp.float32),
    mesh=_mesh,
    scratch_types=[
        pltpu.VMEM_SHARED((NP, H), jnp.float32),
        pltpu.VMEM((1, IW2), jnp.int32),
        pltpu.VMEM((1, IW2), jnp.int32),
        pltpu.VMEM((1, IW2), jnp.int32),
        pltpu.VMEM((1, IW2), jnp.int32),
        pltpu.VMEM((IW2, H), jnp.float32),
        pltpu.VMEM((IW2, H), jnp.float32),
        pltpu.SemaphoreType.DMA,
        pltpu.SemaphoreType.DMA,
        pltpu.SemaphoreType.DMA,
        pltpu.SemaphoreType.DMA,
    ],
)
def _agg_kernel(hs, edges, zeros_hbm, agg_out, acc,
                src0, dst0, src1, dst1, rows0, rows1,
                gsem0, gsem1, ssem0, ssem1):
    c = lax.axis_index("c")
    s = lax.axis_index("s")

    for r in range(R):
        pltpu.sync_copy(zeros_hbm, acc.at[pl.ds(s * RPT, RPT)])
        plsc.subcore_barrier()

        def pair_body(i, _):
            k0 = 2 * i
            k1 = 2 * i + 1

            @pl.when(i > 0)
            def _():
                pltpu.make_async_copy(
                    zeros_hbm.at[pl.ds(0, IW2)], rows0, ssem0).wait()
            pltpu.sync_copy(edges.at[r, 0, s, k0], src0)
            pltpu.sync_copy(edges.at[r, 1, s, k0], dst0)
            g0 = pltpu.async_copy(hs.at[c, r].at[src0.at[0]], rows0, gsem0)

            @pl.when(i > 0)
            def _():
                pltpu.make_async_copy(
                    zeros_hbm.at[pl.ds(0, IW2)], rows1, ssem1).wait()
            pltpu.sync_copy(edges.at[r, 0, s, k1], src1)
            pltpu.sync_copy(edges.at[r, 1, s, k1], dst1)
            g1 = pltpu.async_copy(hs.at[c, r].at[src1.at[0]], rows1, gsem1)

            g0.wait()
            pltpu.async_copy(rows0, acc.at[dst0.at[0]], ssem0, add=True)
            g1.wait()
            pltpu.async_copy(rows1, acc.at[dst1.at[0]], ssem1, add=True)
            return _
        lax.fori_loop(0, NCH2 // 2, pair_body, None)

        pltpu.make_async_copy(zeros_hbm.at[pl.ds(0, IW2)], rows0, ssem0).wait()
        pltpu.sync_copy(edges.at[r, 0, s, NCH2 - 1], src0)
        pltpu.sync_copy(edges.at[r, 1, s, NCH2 - 1], dst0)
        pltpu.async_copy(hs.at[c, r].at[src0.at[0]], rows0, gsem0).wait()
        pltpu.async_copy(rows0, acc.at[dst0.at[0]], ssem0, add=True)

        pltpu.make_async_copy(zeros_hbm.at[pl.ds(0, IW2)], rows0, ssem0).wait()
        pltpu.make_async_copy(zeros_hbm.at[pl.ds(0, IW2)], rows1, ssem1).wait()

        plsc.subcore_barrier()
        pltpu.sync_copy(acc.at[pl.ds(s * RPT, RPT)],
                        agg_out.at[c, r, pl.ds(s * RPT, RPT)])


@functools.partial(
    pl.kernel,
    out_type=[jax.ShapeDtypeStruct((NC, E, H), jnp.float32)
              for _ in range(4)],
    mesh=_mesh,
    scratch_types=[
        pltpu.VMEM((CR, IW), jnp.int32),
        pltpu.VMEM((CR, IW), jnp.int32),
        pltpu.VMEM((CR * IW, H), jnp.float32),
        pltpu.VMEM((CR * IW, H), jnp.float32),
        pltpu.SemaphoreType.DMA,
        pltpu.SemaphoreType.DMA,
        pltpu.SemaphoreType.DMA,
        pltpu.SemaphoreType.DMA,
    ],
)
def _score_gather_kernel(h2s, pe, ne, zeros_hbm, pu, pv, nu, nv,
                         idx0, idx1, rows0, rows1,
                         gsem0, gsem1, wsem0, wsem1):
    c = lax.axis_index("c")
    s = lax.axis_index("s")
    CB = CR * IW

    for eref, io, outref in ((pe, 0, pu), (pe, 1, pv),
                             (ne, 0, nu), (ne, 1, nv)):
        def pair_body(i, _, eref=eref, io=io, outref=outref):
            k0 = 2 * i
            k1 = 2 * i + 1

            @pl.when(i > 0)
            def _():
                pltpu.make_async_copy(
                    zeros_hbm.at[pl.ds(0, CB)], rows0, wsem0).wait()
            pltpu.sync_copy(eref.at[io, s, k0], idx0)
            d0 = [pltpu.async_copy(h2s.at[c].at[idx0.at[j]],
                                   rows0.at[pl.ds(j * IW, IW)], gsem0)
                  for j in range(CR)]

            @pl.when(i > 0)
            def _():
                pltpu.make_async_copy(
                    zeros_hbm.at[pl.ds(0, CB)], rows1, wsem1).wait()
            pltpu.sync_copy(eref.at[io, s, k1], idx1)
            d1 = [pltpu.async_copy(h2s.at[c].at[idx1.at[j]],
                                   rows1.at[pl.ds(j * IW, IW)], gsem1)
                  for j in range(CR)]

            for d in d0:
                d.wait()
            pltpu.async_copy(
                rows0, outref.at[c, pl.ds((s * NCHUNK + k0) * CB, CB)], wsem0)
            for d in d1:
                d.wait()
            pltpu.async_copy(
                rows1, outref.at[c, pl.ds((s * NCHUNK + k1) * CB, CB)], wsem1)
            return _
        lax.fori_loop(0, NCHUNK // 2, pair_body, None)

        pltpu.make_async_copy(zeros_hbm.at[pl.ds(0, CB)], rows0, wsem0).wait()
        pltpu.make_async_copy(zeros_hbm.at[pl.ds(0, CB)], rows1, wsem1).wait()


NB = 256
GRID = NP // NB


def _tc_a_body(x_ref, deg_ref, w1_ref, hs_ref, rs_ref):
    dsum = jnp.sum(deg_ref[...], axis=1)
    deg = jnp.concatenate([dsum[0], dsum[1]], axis=0)
    rs = lax.rsqrt(jnp.maximum(deg, 1.0))
    rs_ref[...] = rs
    x = x_ref[...]
    for r in range(R):
        xs = x * rs[2 * r][:, None]
        t = jnp.dot(xs, w1_ref[r], preferred_element_type=jnp.float32)
        hs_ref[0, r] = t[:, :H]
        hs_ref[1, r] = t[:, H:]


def _tc_b_body(agg_ref, rs_ref, b1_ref, w2_ref, hs2_ref):
    rs = rs_ref[...]
    bsum = b1_ref[0] + b1_ref[1] + b1_ref[2]
    h = jnp.broadcast_to(bsum[None, :], (NB, D))
    for r in range(R):
        ar = jnp.concatenate([agg_ref[0, r], agg_ref[1, r]], axis=1)
        h = h + ar * rs[2 * r + 1][:, None]
    h = jnp.maximum(h, 0.0)
    for r in range(R):
        hsr = h * rs[6 + 2 * r][:, None]
        t = jnp.dot(hsr, w2_ref[r], preferred_element_type=jnp.float32)
        hs2_ref[0, r] = t[:, :H]
        hs2_ref[1, r] = t[:, H:]


def _tc_c_body(agg_ref, rs_ref, b2_ref, h2s_ref):
    rs = rs_ref[...]
    bsum = b2_ref[0] + b2_ref[1] + b2_ref[2]
    h = jnp.broadcast_to(bsum[None, :], (NB, D))
    for r in range(R):
        ar = jnp.concatenate([agg_ref[0, r], agg_ref[1, r]], axis=1)
        h = h + ar * rs[7 + 2 * r][:, None]
    h2s_ref[0] = h[:, :H]
    h2s_ref[1] = h[:, H:]


BE = 4000
DGRID = E // BE
DCOL = 500


def _tc_d_body(pu_ref, pv_ref, nu_ref, nv_ref, pos_ref, neg_ref):
    pos_ref[...] = jnp.sum(pu_ref[0] * pv_ref[0] + pu_ref[1] * pv_ref[1],
                           axis=1).reshape(BE // DCOL, DCOL)
    neg_ref[...] = jnp.sum(nu_ref[0] * nv_ref[0] + nu_ref[1] * nv_ref[1],
                           axis=1).reshape(BE // DCOL, DCOL)


def kernel(x, block1_edges, block2_edges, pos_edges, neg_edges, W1, b1, W2, b2):
    f32 = jnp.float32
    xp = jnp.pad(x, ((0, NP - N), (0, 0)))
    b1d = block1_edges.reshape(R, 2, NT, DCHUNK, DROWS, 16)
    b2d = block2_edges.reshape(R, 2, NT, DCHUNK, DROWS, 16)
    b1r = block1_edges.reshape(R, 2, NT, NCH2, 1, IW2)
    b2r = block2_edges.reshape(R, 2, NT, NCH2, 1, IW2)
    assert NT * NCH2 * IW2 == E
    per = pos_edges.reshape(2, NT, NCHUNK, CR, IW)
    ner = neg_edges.reshape(2, NT, NCHUNK, CR, IW)
    zeros_hbm = jnp.zeros((RPT, H), f32)

    deg = _deg_kernel(b1d, b2d).reshape(NC, NT, 6, NP)

    hs1, rs = pl.pallas_call(
        _tc_a_body,
        grid=(GRID,),
        in_specs=[
            pl.BlockSpec((NB, D), lambda i: (i, 0)),
            pl.BlockSpec((NC, NT, 6, NB), lambda i: (0, 0, 0, i)),
            pl.BlockSpec((R, D, D), lambda i: (0, 0, 0)),
        ],
        out_specs=[
            pl.BlockSpec((NC, R, NB, H), lambda i: (0, 0, i, 0)),
            pl.BlockSpec((12, NB), lambda i: (0, i)),
        ],
        out_shape=[
            jax.ShapeDtypeStruct((NC, R, NP, H), f32),
            jax.ShapeDtypeStruct((12, NP), f32),
        ],
    )(xp, deg, W1)

    agg1 = _agg_kernel(hs1, b1r, zeros_hbm)

    hs2 = pl.pallas_call(
        _tc_b_body,
        grid=(GRID,),
        in_specs=[
            pl.BlockSpec((NC, R, NB, H), lambda i: (0, 0, i, 0)),
            pl.BlockSpec((12, NB), lambda i: (0, i)),
            pl.BlockSpec((R, D), lambda i: (0, 0)),
            pl.BlockSpec((R, D, D), lambda i: (0, 0, 0)),
        ],
        out_specs=pl.BlockSpec((NC, R, NB, H), lambda i: (0, 0, i, 0)),
        out_shape=jax.ShapeDtypeStruct((NC, R, NP, H), f32),
    )(agg1, rs, b1, W2)

    agg2 = _agg_kernel(hs2, b2r, zeros_hbm)

    h2s = pl.pallas_call(
        _tc_c_body,
        grid=(GRID,),
        in_specs=[
            pl.BlockSpec((NC, R, NB, H), lambda i: (0, 0, i, 0)),
            pl.BlockSpec((12, NB), lambda i: (0, i)),
            pl.BlockSpec((R, D), lambda i: (0, 0)),
        ],
        out_specs=pl.BlockSpec((NC, NB, H), lambda i: (0, i, 0)),
        out_shape=jax.ShapeDtypeStruct((NC, NP, H), f32),
    )(agg2, rs, b2)

    pu, pv, nu, nv = _score_gather_kernel(h2s, per, ner, zeros_hbm)

    pos, neg = pl.pallas_call(
        _tc_d_body,
        grid=(DGRID,),
        in_specs=[pl.BlockSpec((NC, BE, H), lambda i: (0, i, 0))
                  for _ in range(4)],
        out_specs=[pl.BlockSpec((BE // DCOL, DCOL), lambda i: (i, 0))
                   for _ in range(2)],
        out_shape=[jax.ShapeDtypeStruct((E // DCOL, DCOL), f32)
                   for _ in range(2)],
    )(pu, pv, nu, nv)

    return (pos.reshape(E, 1), neg.reshape(E, 1))
